# Initial kernel scaffold; baseline (speedup 1.0000x reference)
import jax
import jax.numpy as jnp
from jax.experimental import pallas as pl


def kernel(x, x_res_grid, edge_index, W1, b1, W2, b2, Wl1, bl1, Wl2, bl2):
    raise NotImplementedError("write your pallas kernel here")


# speedup vs baseline: 10.7591x; 10.7591x over previous
import jax
import jax.numpy as jnp
from jax import lax
from jax.experimental import pallas as pl
from jax.experimental.pallas import tpu as pltpu
from jax.experimental.pallas import tpu_sc as plsc

N_GRID = 32580
N_MESH = 17420
N = N_GRID + N_MESH
E = 800000
IN_CH = 48
HID = 256
OUT_CH = 48

NC = 2
NS = 16
BATCH = 128
G = 8

NB16 = 392
EPAD16 = NS * NB16 * BATCH
N_PAD = 50048
ROWS16 = N_PAD // NS

NB32 = 200
EPAD32 = NC * NS * NB32 * BATCH

CH = 32

KAGG = 2
AGG_CHUNK = N_PAD // KAGG
AGG_DUMMY = AGG_CHUNK
AGG_ROWS = AGG_CHUNK + 8
AGG_TROWS = AGG_CHUNK // NS

KDEG = 2
DEG_W = 8
DEG_CHUNK = 25600
DEG_DUMMY = DEG_CHUNK
DEG_ROWS = DEG_CHUNK + 8
DEG_ZROWS = 800

BN = 3128
NBLK = N_PAD // BN


def _sc_mesh():
    return plsc.VectorSubcoreMesh(core_axis_name="c", subcore_axis_name="s")


def _rebase(dst_g, dst_r, base, lim, dummy):
    for b in range(G):
        for k in range(BATCH // 16):
            d = dst_g[b, pl.ds(k * 16, 16)]
            r = d - base
            ok = (r >= 0) & (r < lim)
            dst_r[b, pl.ds(k * 16, 16)] = jnp.where(ok, r, dummy)


def _deg_body(dst_hbm, out0, out1, dst_g, dst_r, ones_v, zbuf, acc):
    c = lax.axis_index("c")
    s = lax.axis_index("s")
    w = c * NS + s

    def fill(i, _):
        ones_v[i, :] = jnp.full((DEG_W,), 1.0, jnp.float32)
        return None
    lax.fori_loop(0, BATCH, fill, None)

    def zfill(i, _):
        zbuf[i, :] = jnp.zeros((DEG_W,), jnp.float32)
        return None
    lax.fori_loop(0, DEG_ZROWS, zfill, None)

    for chunk in range(KDEG):
        base = chunk * DEG_CHUNK
        def zero(k, _):
            pltpu.sync_copy(zbuf, acc.at[pl.ds(s * 1600 + k * DEG_ZROWS,
                                               DEG_ZROWS)])
            return None
        lax.fori_loop(0, 2, zero, None)
        plsc.subcore_barrier()

        def group(g, _):
            pltpu.sync_copy(dst_hbm.at[w, pl.ds(g * G, G)], dst_g)
            _rebase(dst_g, dst_r, base, DEG_CHUNK, DEG_DUMMY)
            for b in range(G):
                pltpu.sync_copy(ones_v, acc.at[dst_r.at[b]], add=True)
            return None
        lax.fori_loop(0, NB32 // G, group, None)

        plsc.subcore_barrier()
        nrows = 1600 if chunk == 0 else (N_PAD - DEG_CHUNK) // NS
        for out, cc in ((out0, 0), (out1, 1)):
            @pl.when(c == cc)
            def _(out=out):
                pltpu.sync_copy(
                    acc.at[pl.ds(s * nrows, nrows)],
                    out.at[pl.ds(base + s * nrows, nrows)])
        plsc.subcore_barrier()


def _make_deg_kernel():
    return pl.kernel(
        _deg_body,
        out_type=[jax.ShapeDtypeStruct((N_PAD, DEG_W), jnp.float32),
                  jax.ShapeDtypeStruct((N_PAD, DEG_W), jnp.float32)],
        mesh=_sc_mesh(),
        scratch_types=[
            pltpu.VMEM((G, BATCH), jnp.int32),
            pltpu.VMEM((G, BATCH), jnp.int32),
            pltpu.VMEM((BATCH, DEG_W), jnp.float32),
            pltpu.VMEM((DEG_ZROWS, DEG_W), jnp.float32),
            pltpu.VMEM_SHARED((DEG_ROWS, DEG_W), jnp.float32),
        ],
        compiler_params=pltpu.CompilerParams(use_tc_tiling_on_sc=False),
    )


def _agg_body(tab0, tab1, src_hbm, dst_hbm, out0, out1,
              src_g, dst_g, dst_r, gbuf, acc):
    c = lax.axis_index("c")
    s = lax.axis_index("s")

    def run(tab, out):
        for chunk in range(KAGG):
            base = chunk * AGG_CHUNK
            pltpu.sync_copy(
                tab.at[pl.ds(base + s * AGG_TROWS, AGG_TROWS)],
                acc.at[pl.ds(s * AGG_TROWS, AGG_TROWS)])
            plsc.subcore_barrier()

            def group(g, _):
                pltpu.sync_copy(src_hbm.at[s, pl.ds(g * G, G)], src_g)
                pltpu.sync_copy(dst_hbm.at[s, pl.ds(g * G, G)], dst_g)
                _rebase(dst_g, dst_r, base, AGG_CHUNK, AGG_DUMMY)
                for b in range(G):
                    pltpu.sync_copy(tab.at[src_g.at[b]], gbuf)
                    pltpu.sync_copy(gbuf, acc.at[dst_r.at[b]], add=True)
                return None
            lax.fori_loop(0, NB16 // G, group, None)

            plsc.subcore_barrier()
            pltpu.sync_copy(
                acc.at[pl.ds(s * AGG_TROWS, AGG_TROWS)],
                out.at[pl.ds(base + s * AGG_TROWS, AGG_TROWS)])
            plsc.subcore_barrier()

    @pl.when(c == 0)
    def _():
        run(tab0, out0)

    @pl.when(c == 1)
    def _():
        run(tab1, out1)


def _make_agg_kernel():
    return pl.kernel(
        _agg_body,
        out_type=[jax.ShapeDtypeStruct((N_PAD, CH), jnp.float32),
                  jax.ShapeDtypeStruct((N_PAD, CH), jnp.float32)],
        mesh=_sc_mesh(),
        scratch_types=[
            pltpu.VMEM((G, BATCH), jnp.int32),
            pltpu.VMEM((G, BATCH), jnp.int32),
            pltpu.VMEM((G, BATCH), jnp.int32),
            pltpu.VMEM((BATCH, CH), jnp.float32),
            pltpu.VMEM_SHARED((AGG_ROWS, CH), jnp.float32),
        ],
        compiler_params=pltpu.CompilerParams(use_tc_tiling_on_sc=False),
    )


def _fold_w_body(w2, wl1, b2, bl1, wl2, bl2, wc, bc):
    a = jnp.dot(wl1[...], wl2[...], preferred_element_type=jnp.float32)
    wc[...] = jnp.dot(w2[...], a, preferred_element_type=jnp.float32)
    bc[...] = (jnp.dot(b2[...], a, preferred_element_type=jnp.float32)
               + jnp.dot(bl1[...], wl2[...], preferred_element_type=jnp.float32)
               + bl2[...])


def _fold_weights(W2, Wl1, b2, bl1, Wl2, bl2):
    return pl.pallas_call(
        _fold_w_body,
        out_shape=[jax.ShapeDtypeStruct((HID, OUT_CH), jnp.float32),
                   jax.ShapeDtypeStruct((1, OUT_CH), jnp.float32)],
    )(W2, Wl1, b2.reshape(1, HID), bl1.reshape(1, HID), Wl2,
      bl2.reshape(1, OUT_CH))


def _prep_body(dp0, dp1, h0, dinv, u_lo, u_hi):
    deg = 1.0 + dp0[:, 0:1] + dp1[:, 0:1]
    di = lax.rsqrt(deg)
    dinv[...] = di
    u = h0[...] * di
    u_lo[...] = u[:, :CH]
    u_hi[...] = jnp.concatenate(
        [u[:, CH:], jnp.zeros((BN, 2 * CH - IN_CH), jnp.float32)], axis=1)


def _prep_nodes(dp0, dp1, h0):
    return pl.pallas_call(
        _prep_body,
        grid=(NBLK,),
        in_specs=[pl.BlockSpec((BN, DEG_W), lambda i: (i, 0)),
                  pl.BlockSpec((BN, DEG_W), lambda i: (i, 0)),
                  pl.BlockSpec((BN, IN_CH), lambda i: (i, 0))],
        out_specs=[pl.BlockSpec((BN, 1), lambda i: (i, 0)),
                   pl.BlockSpec((BN, CH), lambda i: (i, 0)),
                   pl.BlockSpec((BN, CH), lambda i: (i, 0))],
        out_shape=[jax.ShapeDtypeStruct((N_PAD, 1), jnp.float32),
                   jax.ShapeDtypeStruct((N_PAD, CH), jnp.float32),
                   jax.ShapeDtypeStruct((N_PAD, CH), jnp.float32)],
    )(dp0, dp1, h0)


def _mid_body(s_lo, s_hi, dinv, w1, b1, wc, u_lo, u_hi):
    di = dinv[...]
    agg = jnp.concatenate([s_lo[...], s_hi[:, :IN_CH - CH]], axis=1) * di
    z = jnp.dot(agg, w1[...], preferred_element_type=jnp.float32) + b1[...]
    g = 0.5 * z * (1.0 + lax.erf(z * 0.7071067811865476))
    t = jnp.dot(g, wc[...], preferred_element_type=jnp.float32) * di
    u_lo[...] = t[:, :CH]
    u_hi[...] = jnp.concatenate(
        [t[:, CH:], jnp.zeros((BN, 2 * CH - IN_CH), jnp.float32)], axis=1)


def _mid_nodes(s_lo, s_hi, dinv, W1, b1, Wc):
    return pl.pallas_call(
        _mid_body,
        grid=(NBLK,),
        in_specs=[pl.BlockSpec((BN, CH), lambda i: (i, 0)),
                  pl.BlockSpec((BN, CH), lambda i: (i, 0)),
                  pl.BlockSpec((BN, 1), lambda i: (i, 0)),
                  pl.BlockSpec((IN_CH, HID), lambda i: (0, 0)),
                  pl.BlockSpec((1, HID), lambda i: (0, 0)),
                  pl.BlockSpec((HID, OUT_CH), lambda i: (0, 0))],
        out_specs=[pl.BlockSpec((BN, CH), lambda i: (i, 0)),
                   pl.BlockSpec((BN, CH), lambda i: (i, 0))],
        out_shape=[jax.ShapeDtypeStruct((N_PAD, CH), jnp.float32),
                   jax.ShapeDtypeStruct((N_PAD, CH), jnp.float32)],
    )(s_lo, s_hi, dinv, W1, b1.reshape(1, HID), Wc)


def _final_body(s_lo, s_hi, dinv, bc, y):
    agg = jnp.concatenate([s_lo[...], s_hi[:, :IN_CH - CH]], axis=1)
    y[...] = agg * dinv[...] + bc[...]


def _final_nodes(s_lo, s_hi, dinv, bc):
    return pl.pallas_call(
        _final_body,
        grid=(NBLK,),
        in_specs=[pl.BlockSpec((BN, CH), lambda i: (i, 0)),
                  pl.BlockSpec((BN, CH), lambda i: (i, 0)),
                  pl.BlockSpec((BN, 1), lambda i: (i, 0)),
                  pl.BlockSpec((1, OUT_CH), lambda i: (0, 0))],
        out_specs=pl.BlockSpec((BN, OUT_CH), lambda i: (i, 0)),
        out_shape=jax.ShapeDtypeStruct((N_PAD, OUT_CH), jnp.float32),
    )(s_lo, s_hi, dinv, bc)


@jax.jit
def kernel(x, x_res_grid, edge_index, W1, b1, W2, b2, Wl1, bl1, Wl2, bl2):
    src = edge_index[0]
    dst = edge_index[1]

    src16 = jnp.concatenate(
        [src, jnp.zeros((EPAD16 - E,), jnp.int32)]).reshape(NS, NB16, BATCH)
    dst16 = jnp.concatenate(
        [dst, jnp.full((EPAD16 - E,), N, jnp.int32)]).reshape(NS, NB16, BATCH)
    dst32 = jnp.concatenate(
        [dst, jnp.full((EPAD32 - E,), N, jnp.int32)]).reshape(NC * NS, NB32, BATCH)

    h0 = jnp.concatenate([x_res_grid[0], x[0]], axis=-1).T
    h0 = jnp.pad(h0, ((0, N_PAD - N), (0, 0)))

    deg0, deg1 = _make_deg_kernel()(dst32)
    Wc, bc = _fold_weights(W2, Wl1, b2, bl1, Wl2, bl2)
    dinv, u0_lo, u0_hi = _prep_nodes(deg0, deg1, h0)

    agg = _make_agg_kernel()
    s1_lo, s1_hi = agg(u0_lo, u0_hi, src16, dst16)
    u1_lo, u1_hi = _mid_nodes(s1_lo, s1_hi, dinv, W1, b1, Wc)
    s2_lo, s2_hi = agg(u1_lo, u1_hi, src16, dst16)
    y = _final_nodes(s2_lo, s2_hi, dinv, bc)

    return y[:N_GRID].T[None]
